# baseline (speedup 1.0000x reference)
import functools

import jax
import jax.numpy as jnp
from jax import lax
from jax.experimental import pallas as pl
from jax.experimental.pallas import tpu as pltpu
from jax.experimental.pallas import tpu_sc as plsc

N = 50000
E = 800000
IN_FEAT = 128
HIDDEN = 64
OUT_FEAT = 128

BLK = 1024
GRID = (N + BLK - 1) // BLK
N_PAD = GRID * BLK

NC = 2
NS = 16
HALF = N_PAD // NC
ROWS_PER_TILE = HALF // NS
EDGE_CHUNK = 128
NCHUNKS = E // EDGE_CHUNK
CPT = 396
E_PAD = (NS * CPT + 2) * EDGE_CHUNK


def _embed_ln_body(feat_ref, coord_ref, wf_ref, bf_ref, wc_ref, bc_ref,
                   gamma_ref, beta_ref, x_ref):
    f = feat_ref[...]
    fe = jnp.maximum(
        jnp.dot(f, wf_ref[...], preferred_element_type=jnp.float32)
        + bf_ref[...], 0.0)
    c = coord_ref[...]
    ce = jnp.maximum(
        c[:, 0:1] * wc_ref[0:1, :] + c[:, 1:2] * wc_ref[1:2, :] + bc_ref[...],
        0.0)
    x = fe + ce
    mu = jnp.mean(x, axis=-1, keepdims=True)
    var = jnp.mean((x - mu) ** 2, axis=-1, keepdims=True)
    y = (x - mu) * lax.rsqrt(var + 1e-5) * gamma_ref[...] + beta_ref[...]
    x_ref[...] = y


def _embed_ln(features, coordinates, W_feat, b_feat, W_coord, b_coord,
              gamma, beta):
    return pl.pallas_call(
        _embed_ln_body,
        grid=(GRID,),
        in_specs=[
            pl.BlockSpec((BLK, IN_FEAT), lambda i: (i, 0)),
            pl.BlockSpec((BLK, 2), lambda i: (i, 0)),
            pl.BlockSpec((IN_FEAT, HIDDEN), lambda i: (0, 0)),
            pl.BlockSpec((1, HIDDEN), lambda i: (0, 0)),
            pl.BlockSpec((2, HIDDEN), lambda i: (0, 0)),
            pl.BlockSpec((1, HIDDEN), lambda i: (0, 0)),
            pl.BlockSpec((1, HIDDEN), lambda i: (0, 0)),
            pl.BlockSpec((1, HIDDEN), lambda i: (0, 0)),
        ],
        out_specs=pl.BlockSpec((BLK, HIDDEN), lambda i: (i, 0)),
        out_shape=jax.ShapeDtypeStruct((N, HIDDEN), jnp.float32),
    )(features, coordinates, W_feat, b_feat.reshape(1, HIDDEN),
      W_coord, b_coord.reshape(1, HIDDEN), gamma.reshape(1, HIDDEN),
      beta.reshape(1, HIDDEN))


def _spmm_body(x_hbm, idx_hbm, val_hbm, agg_hbm, agg_sp, xb, ibuf, vbuf, rlb,
               sem_i, sem_g, sem_s):
    cid = lax.axis_index("c")
    sid = lax.axis_index("s")
    base_row = cid * HALF
    tbase = sid * CPT
    row0 = sid * ROWS_PER_TILE

    def _zrow(r, _):
        for d in range(HIDDEN // 16):
            xb[0, r, pl.ds(d * 16, 16)] = jnp.zeros((16,), jnp.float32)
        return 0
    lax.fori_loop(0, EDGE_CHUNK, _zrow, 0)
    nfull = ROWS_PER_TILE // EDGE_CHUNK
    rem = ROWS_PER_TILE - nfull * EDGE_CHUNK
    for k in range(nfull):
        pltpu.sync_copy(xb.at[0],
                        agg_sp.at[pl.ds(row0 + k * EDGE_CHUNK, EDGE_CHUNK)])
    pltpu.sync_copy(xb.at[0, pl.ds(0, rem)],
                    agg_sp.at[pl.ds(row0 + nfull * EDGE_CHUNK, rem)])
    plsc.subcore_barrier()

    def idx_pair(j, b):
        return idx_hbm.at[:, pl.ds(j * EDGE_CHUNK, EDGE_CHUNK)], ibuf.at[b]

    def val_pair(j, b):
        return val_hbm.at[pl.ds(j * EDGE_CHUNK, EDGE_CHUNK)], vbuf.at[b]

    def issue_idx(j, b):
        s, d = idx_pair(j, b)
        pltpu.async_copy(s, d, sem_i.at[b])
        s, d = val_pair(j, b)
        pltpu.async_copy(s, d, sem_i.at[b])

    def drain_idx(j, b):
        s, d = idx_pair(j, b)
        pltpu.make_async_copy(s, d, sem_i.at[b]).wait()
        s, d = val_pair(j, b)
        pltpu.make_async_copy(s, d, sem_i.at[b]).wait()

    def gat_pair(b):
        return x_hbm.at[ibuf.at[b, 1]], xb.at[b]

    def sct_pair(b):
        return xb.at[b], agg_sp.at[rlb.at[b]]

    issue_idx(tbase + 0, 0)
    issue_idx(tbase + 1, 1)
    drain_idx(tbase + 0, 0)
    s, d = gat_pair(0)
    pltpu.async_copy(s, d, sem_g.at[0])

    def pipe_body(i, _):
        for p in range(3):
            jl = i * 3 + p
            j = tbase + jl
            pn = (p + 1) % 3
            pp = (p + 2) % 3
            drain_idx(j + 1, pn)
            @pl.when(jl >= 2)
            def _drain_scatter():
                s2, d2 = sct_pair(pn)
                pltpu.make_async_copy(s2, d2, sem_s.at[pn]).wait()
            s, d = gat_pair(pn)
            pltpu.async_copy(s, d, sem_g.at[pn])
            issue_idx(j + 2, pp)
            s, d = gat_pair(p)
            pltpu.make_async_copy(s, d, sem_g.at[p]).wait()

            def gbody(g, _):
                r = ibuf[p, 0, pl.ds(g * 16, 16)]
                v = vbuf[p, pl.ds(g * 16, 16)]
                rl = r - base_row
                inr = (rl >= 0) & (rl < HALF)
                rlb[p, pl.ds(g * 16, 16)] = jnp.where(inr, rl, 0)
                w = jnp.where(inr, v, 0.0)
                e_idx = g * 16 + lax.iota(jnp.int32, 16)
                xp = xb.at[p]
                for dd in range(HIDDEN):
                    d_idx = jnp.full((16,), dd, jnp.int32)
                    vals = plsc.load_gather(xp, [e_idx, d_idx])
                    plsc.store_scatter(xp, [e_idx, d_idx], vals * w)
                return 0
            lax.fori_loop(0, EDGE_CHUNK // 16, gbody, 0)
            s, d = sct_pair(p)
            pltpu.async_copy(s, d, sem_s.at[p], add=True)
        return 0

    lax.fori_loop(0, CPT // 3, pipe_body, 0)

    s, d = gat_pair(0)
    pltpu.make_async_copy(s, d, sem_g.at[0]).wait()
    drain_idx(tbase + CPT + 1, 1)
    s, d = sct_pair(1)
    pltpu.make_async_copy(s, d, sem_s.at[1]).wait()
    s, d = sct_pair(2)
    pltpu.make_async_copy(s, d, sem_s.at[2]).wait()

    plsc.subcore_barrier()

    for k in range(nfull):
        off = row0 + k * EDGE_CHUNK
        pltpu.sync_copy(agg_sp.at[pl.ds(off, EDGE_CHUNK)],
                        agg_hbm.at[pl.ds(base_row + off, EDGE_CHUNK)])
    off = row0 + nfull * EDGE_CHUNK
    pltpu.sync_copy(agg_sp.at[pl.ds(off, rem)],
                    agg_hbm.at[pl.ds(base_row + off, rem)])


def _spmm(x, idx_all, val_pad):
    mesh = plsc.VectorSubcoreMesh(core_axis_name="c", subcore_axis_name="s")
    return pl.kernel(
        _spmm_body,
        out_type=jax.ShapeDtypeStruct((N_PAD, HIDDEN), jnp.float32),
        mesh=mesh,
        scratch_types=[
            pltpu.VMEM_SHARED((HALF, HIDDEN), jnp.float32),
            pltpu.VMEM((3, EDGE_CHUNK, HIDDEN), jnp.float32),
            pltpu.VMEM((3, 2, EDGE_CHUNK), jnp.int32),
            pltpu.VMEM((3, EDGE_CHUNK), jnp.float32),
            pltpu.VMEM((3, EDGE_CHUNK), jnp.int32),
            pltpu.SemaphoreType.DMA((3,)),
            pltpu.SemaphoreType.DMA((3,)),
            pltpu.SemaphoreType.DMA((3,)),
        ],
        compiler_params=pltpu.CompilerParams(use_tc_tiling_on_sc=False, needs_layout_passes=False),
    )(x, idx_all, val_pad)


def _out_proj_body(agg_ref, w_ref, out_ref):
    out_ref[...] = jnp.maximum(
        jnp.dot(agg_ref[...], w_ref[...], preferred_element_type=jnp.float32),
        0.0)


def _out_proj(agg, gcn_weight):
    return pl.pallas_call(
        _out_proj_body,
        grid=(GRID,),
        in_specs=[
            pl.BlockSpec((BLK, HIDDEN), lambda i: (i, 0)),
            pl.BlockSpec((HIDDEN, OUT_FEAT), lambda i: (0, 0)),
        ],
        out_specs=pl.BlockSpec((BLK, OUT_FEAT), lambda i: (i, 0)),
        out_shape=jax.ShapeDtypeStruct((N, OUT_FEAT), jnp.float32),
    )(agg, gcn_weight)


@jax.jit
def kernel(features, coordinates, adj_indices, adj_values, W_feat, b_feat,
           W_coord, b_coord, gamma, beta, gcn_weight):
    x = _embed_ln(features, coordinates, W_feat, b_feat, W_coord, b_coord,
                  gamma, beta)
    idx_all = jnp.zeros((2, E_PAD), jnp.int32)
    idx_all = idx_all.at[:, :E].set(adj_indices)
    val_pad = jnp.zeros((E_PAD,), jnp.float32).at[:E].set(adj_values)
    agg = _spmm(x, idx_all, val_pad)
    out = _out_proj(agg, gcn_weight)
    return out


# speedup vs baseline: 4.3779x; 4.3779x over previous
import functools

import jax
import jax.numpy as jnp
from jax import lax
from jax.experimental import pallas as pl
from jax.experimental.pallas import tpu as pltpu
from jax.experimental.pallas import tpu_sc as plsc

N = 50000
E = 800000
IN_FEAT = 128
HIDDEN = 64
OUT_FEAT = 128

BLK = 1024
GRID = (N + BLK - 1) // BLK
N_PAD = GRID * BLK

NC = 2
NS = 16
HALF = N_PAD // NC
ROWS_PER_TILE = HALF // NS
EDGE_CHUNK = 112
CPT = 448
E_PAD = (NS * CPT + 4) * EDGE_CHUNK
DEPTH = 4


def _embed_ln_body(feat_ref, coord_ref, wf_ref, bf_ref, wc_ref, bc_ref,
                   gamma_ref, beta_ref, x_ref):
    f = feat_ref[...]
    fe = jnp.maximum(
        jnp.dot(f, wf_ref[...], preferred_element_type=jnp.float32)
        + bf_ref[...], 0.0)
    c = coord_ref[...]
    ce = jnp.maximum(
        c[:, 0:1] * wc_ref[0:1, :] + c[:, 1:2] * wc_ref[1:2, :] + bc_ref[...],
        0.0)
    x = fe + ce
    mu = jnp.mean(x, axis=-1, keepdims=True)
    var = jnp.mean((x - mu) ** 2, axis=-1, keepdims=True)
    y = (x - mu) * lax.rsqrt(var + 1e-5) * gamma_ref[...] + beta_ref[...]
    x_ref[...] = y


def _embed_ln(features, coordinates, W_feat, b_feat, W_coord, b_coord,
              gamma, beta):
    return pl.pallas_call(
        _embed_ln_body,
        grid=(GRID,),
        in_specs=[
            pl.BlockSpec((BLK, IN_FEAT), lambda i: (i, 0)),
            pl.BlockSpec((BLK, 2), lambda i: (i, 0)),
            pl.BlockSpec((IN_FEAT, HIDDEN), lambda i: (0, 0)),
            pl.BlockSpec((1, HIDDEN), lambda i: (0, 0)),
            pl.BlockSpec((2, HIDDEN), lambda i: (0, 0)),
            pl.BlockSpec((1, HIDDEN), lambda i: (0, 0)),
            pl.BlockSpec((1, HIDDEN), lambda i: (0, 0)),
            pl.BlockSpec((1, HIDDEN), lambda i: (0, 0)),
        ],
        out_specs=pl.BlockSpec((BLK, HIDDEN), lambda i: (i, 0)),
        out_shape=jax.ShapeDtypeStruct((N, HIDDEN), jnp.float32),
    )(features, coordinates, W_feat, b_feat.reshape(1, HIDDEN),
      W_coord, b_coord.reshape(1, HIDDEN), gamma.reshape(1, HIDDEN),
      beta.reshape(1, HIDDEN))


def _spmm_body(x_hbm, idx_hbm, val_hbm, agg_hbm, agg_sp, xb, ibuf, vbuf, rlb,
               sem_i, sem_g, sem_s):
    cid = lax.axis_index("c")
    sid = lax.axis_index("s")
    base_row = cid * HALF
    tbase = sid * CPT
    row0 = sid * ROWS_PER_TILE

    def _zrow(r, _):
        for d in range(HIDDEN // 16):
            xb[0, r, pl.ds(d * 16, 16)] = jnp.zeros((16,), jnp.float32)
        return 0
    lax.fori_loop(0, EDGE_CHUNK, _zrow, 0)
    nfull = ROWS_PER_TILE // EDGE_CHUNK
    for k in range(nfull):
        pltpu.sync_copy(xb.at[0],
                        agg_sp.at[pl.ds(row0 + k * EDGE_CHUNK, EDGE_CHUNK)])
    plsc.subcore_barrier()

    def idx_pair(j, b):
        return idx_hbm.at[:, pl.ds(j * EDGE_CHUNK, EDGE_CHUNK)], ibuf.at[b]

    def val_pair(j, b):
        return val_hbm.at[pl.ds(j * EDGE_CHUNK, EDGE_CHUNK)], vbuf.at[b]

    def issue_idx(j, b):
        s, d = idx_pair(j, b)
        pltpu.async_copy(s, d, sem_i.at[b])
        s, d = val_pair(j, b)
        pltpu.async_copy(s, d, sem_i.at[b])

    def drain_idx(j, b):
        s, d = idx_pair(j, b)
        pltpu.make_async_copy(s, d, sem_i.at[b]).wait()
        s, d = val_pair(j, b)
        pltpu.make_async_copy(s, d, sem_i.at[b]).wait()

    def gat_pair(b):
        return x_hbm.at[ibuf.at[b, 1]], xb.at[b]

    def sct_pair(b):
        return xb.at[b], agg_sp.at[rlb.at[b]]

    for b in range(DEPTH):
        issue_idx(tbase + b, b)
    for b in range(2):
        drain_idx(tbase + b, b)
        s, d = gat_pair(b)
        pltpu.async_copy(s, d, sem_g.at[b])

    def compute_chunk(p):
        def gbody(g, _):
            r = ibuf[p, 0, pl.ds(g * 16, 16)]
            v = vbuf[p, pl.ds(g * 16, 16)]
            rl = r - base_row
            inr = (rl >= 0) & (rl < HALF)
            rlb[p, pl.ds(g * 16, 16)] = jnp.where(inr, rl, 0)
            w = jnp.where(inr, v, 0.0)
            for l in range(16):
                ws = w[l]
                e = g * 16 + l
                for dd in range(HIDDEN // 16):
                    xb[p, e, pl.ds(dd * 16, 16)] = (
                        xb[p, e, pl.ds(dd * 16, 16)] * ws)
            return 0
        lax.fori_loop(0, EDGE_CHUNK // 16, gbody, 0)

    def pipe_body(i, _):
        for p in range(DEPTH):
            jl = i * DEPTH + p
            j = tbase + jl
            p2 = (p + 2) % DEPTH
            drain_idx(j + 2, p2)
            @pl.when(jl >= 2)
            def _drain_scatter():
                s2, d2 = sct_pair(p2)
                pltpu.make_async_copy(s2, d2, sem_s.at[p2]).wait()
            s, d = gat_pair(p2)
            pltpu.async_copy(s, d, sem_g.at[p2])
            s, d = gat_pair(p)
            pltpu.make_async_copy(s, d, sem_g.at[p]).wait()
            compute_chunk(p)
            issue_idx(j + DEPTH, p)
            s, d = sct_pair(p)
            pltpu.async_copy(s, d, sem_s.at[p], add=True)
        return 0

    lax.fori_loop(0, CPT // DEPTH, pipe_body, 0)

    for j in (CPT, CPT + 1):
        s, d = gat_pair(j % DEPTH)
        pltpu.make_async_copy(s, d, sem_g.at[j % DEPTH]).wait()
    for j in (CPT + 2, CPT + 3):
        drain_idx(tbase + j, j % DEPTH)
    for j in (CPT - 2, CPT - 1):
        s, d = sct_pair(j % DEPTH)
        pltpu.make_async_copy(s, d, sem_s.at[j % DEPTH]).wait()

    plsc.subcore_barrier()

    for k in range(nfull):
        off = row0 + k * EDGE_CHUNK
        pltpu.sync_copy(agg_sp.at[pl.ds(off, EDGE_CHUNK)],
                        agg_hbm.at[pl.ds(base_row + off, EDGE_CHUNK)])


def _spmm(x, idx_all, val_pad):
    mesh = plsc.VectorSubcoreMesh(core_axis_name="c", subcore_axis_name="s")
    return pl.kernel(
        _spmm_body,
        out_type=jax.ShapeDtypeStruct((N_PAD, HIDDEN), jnp.float32),
        mesh=mesh,
        scratch_types=[
            pltpu.VMEM_SHARED((HALF, HIDDEN), jnp.float32),
            pltpu.VMEM((DEPTH, EDGE_CHUNK, HIDDEN), jnp.float32),
            pltpu.VMEM((DEPTH, 2, EDGE_CHUNK), jnp.int32),
            pltpu.VMEM((DEPTH, EDGE_CHUNK), jnp.float32),
            pltpu.VMEM((DEPTH, EDGE_CHUNK), jnp.int32),
            pltpu.SemaphoreType.DMA((DEPTH,)),
            pltpu.SemaphoreType.DMA((DEPTH,)),
            pltpu.SemaphoreType.DMA((DEPTH,)),
        ],
        compiler_params=pltpu.CompilerParams(use_tc_tiling_on_sc=False),
    )(x, idx_all, val_pad)


def _out_proj_body(agg_ref, w_ref, out_ref):
    out_ref[...] = jnp.maximum(
        jnp.dot(agg_ref[...], w_ref[...], preferred_element_type=jnp.float32),
        0.0)


def _out_proj(agg, gcn_weight):
    return pl.pallas_call(
        _out_proj_body,
        grid=(GRID,),
        in_specs=[
            pl.BlockSpec((BLK, HIDDEN), lambda i: (i, 0)),
            pl.BlockSpec((HIDDEN, OUT_FEAT), lambda i: (0, 0)),
        ],
        out_specs=pl.BlockSpec((BLK, OUT_FEAT), lambda i: (i, 0)),
        out_shape=jax.ShapeDtypeStruct((N, OUT_FEAT), jnp.float32),
    )(agg, gcn_weight)


@jax.jit
def kernel(features, coordinates, adj_indices, adj_values, W_feat, b_feat,
           W_coord, b_coord, gamma, beta, gcn_weight):
    x = _embed_ln(features, coordinates, W_feat, b_feat, W_coord, b_coord,
                  gamma, beta)
    idx_all = jnp.zeros((2, E_PAD), jnp.int32)
    idx_all = idx_all.at[:, :E].set(adj_indices)
    val_pad = jnp.zeros((E_PAD,), jnp.float32).at[:E].set(adj_values)
    agg = _spmm(x, idx_all, val_pad)
    out = _out_proj(agg, gcn_weight)
    return out


# speedup vs baseline: 4.8510x; 1.1081x over previous
import functools

import jax
import jax.numpy as jnp
from jax import lax
from jax.experimental import pallas as pl
from jax.experimental.pallas import tpu as pltpu
from jax.experimental.pallas import tpu_sc as plsc

N = 50000
E = 800000
IN_FEAT = 128
HIDDEN = 64
OUT_FEAT = 128

BLK = 1024
GRID = (N + BLK - 1) // BLK
N_PAD = GRID * BLK

NC = 2
NS = 16
HALF = N_PAD // NC
ROWS_PER_TILE = HALF // NS
EDGE_CHUNK = 112
CPT = 448
E_PAD = (NS * CPT + 4) * EDGE_CHUNK
DEPTH = 4


def _embed_ln_body(feat_ref, coord_ref, wf_ref, bf_ref, wc_ref, bc_ref,
                   gamma_ref, beta_ref, x_ref):
    f = feat_ref[...]
    fe = jnp.maximum(
        jnp.dot(f, wf_ref[...], preferred_element_type=jnp.float32)
        + bf_ref[...], 0.0)
    c = coord_ref[...]
    ce = jnp.maximum(
        c[:, 0:1] * wc_ref[0:1, :] + c[:, 1:2] * wc_ref[1:2, :] + bc_ref[...],
        0.0)
    x = fe + ce
    mu = jnp.mean(x, axis=-1, keepdims=True)
    var = jnp.mean((x - mu) ** 2, axis=-1, keepdims=True)
    y = (x - mu) * lax.rsqrt(var + 1e-5) * gamma_ref[...] + beta_ref[...]
    x_ref[...] = y


def _embed_ln(features, coordinates, W_feat, b_feat, W_coord, b_coord,
              gamma, beta):
    return pl.pallas_call(
        _embed_ln_body,
        grid=(GRID,),
        in_specs=[
            pl.BlockSpec((BLK, IN_FEAT), lambda i: (i, 0)),
            pl.BlockSpec((BLK, 2), lambda i: (i, 0)),
            pl.BlockSpec((IN_FEAT, HIDDEN), lambda i: (0, 0)),
            pl.BlockSpec((1, HIDDEN), lambda i: (0, 0)),
            pl.BlockSpec((2, HIDDEN), lambda i: (0, 0)),
            pl.BlockSpec((1, HIDDEN), lambda i: (0, 0)),
            pl.BlockSpec((1, HIDDEN), lambda i: (0, 0)),
            pl.BlockSpec((1, HIDDEN), lambda i: (0, 0)),
        ],
        out_specs=pl.BlockSpec((BLK, HIDDEN), lambda i: (i, 0)),
        out_shape=jax.ShapeDtypeStruct((N, HIDDEN), jnp.float32),
    )(features, coordinates, W_feat, b_feat.reshape(1, HIDDEN),
      W_coord, b_coord.reshape(1, HIDDEN), gamma.reshape(1, HIDDEN),
      beta.reshape(1, HIDDEN))


def _spmm_body(x_hbm, idx_hbm, val_hbm, agg_hbm, agg_sp, xb, ibuf, vbuf, rlb,
               sem_i, sem_g, sem_s):
    cid = lax.axis_index("c")
    sid = lax.axis_index("s")
    base_row = cid * HALF
    tbase = sid * CPT
    row0 = sid * ROWS_PER_TILE

    def _zrow(r, _):
        for d in range(HIDDEN // 16):
            xb[0, r, pl.ds(d * 16, 16)] = jnp.zeros((16,), jnp.float32)
        return 0
    lax.fori_loop(0, EDGE_CHUNK, _zrow, 0)
    nfull = ROWS_PER_TILE // EDGE_CHUNK
    for k in range(nfull):
        pltpu.sync_copy(xb.at[0],
                        agg_sp.at[pl.ds(row0 + k * EDGE_CHUNK, EDGE_CHUNK)])
    plsc.subcore_barrier()

    def idx_pair(j, b):
        return idx_hbm.at[:, pl.ds(j * EDGE_CHUNK, EDGE_CHUNK)], ibuf.at[b]

    def val_pair(j, b):
        return val_hbm.at[pl.ds(j * EDGE_CHUNK, EDGE_CHUNK)], vbuf.at[b]

    def issue_idx(j, b):
        s, d = idx_pair(j, b)
        pltpu.async_copy(s, d, sem_i.at[b])
        s, d = val_pair(j, b)
        pltpu.async_copy(s, d, sem_i.at[b])

    def drain_idx(j, b):
        s, d = idx_pair(j, b)
        pltpu.make_async_copy(s, d, sem_i.at[b]).wait()
        s, d = val_pair(j, b)
        pltpu.make_async_copy(s, d, sem_i.at[b]).wait()

    def gat_pair(b):
        return x_hbm.at[ibuf.at[b, 1]], xb.at[b]

    def sct_pair(b):
        return xb.at[b], agg_sp.at[rlb.at[b]]

    for b in range(DEPTH):
        issue_idx(tbase + b, b)
    for b in range(2):
        drain_idx(tbase + b, b)
        s, d = gat_pair(b)
        pltpu.async_copy(s, d, sem_g.at[b])

    def compute_chunk(p):
        @plsc.parallel_loop(0, EDGE_CHUNK // 16, 1, unroll=2)
        def gbody(g):
            r = ibuf[p, 0, pl.ds(g * 16, 16)]
            v = vbuf[p, pl.ds(g * 16, 16)]
            rl = r - base_row
            inr = (rl >= 0) & (rl < HALF)
            rlb[p, pl.ds(g * 16, 16)] = jnp.where(inr, rl, 0)
            w = jnp.where(inr, v, 0.0)
            for l in range(16):
                ws = w[l]
                e = g * 16 + l
                for dd in range(HIDDEN // 16):
                    xb[p, e, pl.ds(dd * 16, 16)] = (
                        xb[p, e, pl.ds(dd * 16, 16)] * ws)

    def pipe_body(i, _):
        for p in range(DEPTH):
            jl = i * DEPTH + p
            j = tbase + jl
            p2 = (p + 2) % DEPTH
            drain_idx(j + 2, p2)
            @pl.when(jl >= 2)
            def _drain_scatter():
                s2, d2 = sct_pair(p2)
                pltpu.make_async_copy(s2, d2, sem_s.at[p2]).wait()
            s, d = gat_pair(p2)
            pltpu.async_copy(s, d, sem_g.at[p2])
            s, d = gat_pair(p)
            pltpu.make_async_copy(s, d, sem_g.at[p]).wait()
            compute_chunk(p)
            issue_idx(j + DEPTH, p)
            s, d = sct_pair(p)
            pltpu.async_copy(s, d, sem_s.at[p], add=True)
        return 0

    lax.fori_loop(0, CPT // DEPTH, pipe_body, 0)

    for j in (CPT, CPT + 1):
        s, d = gat_pair(j % DEPTH)
        pltpu.make_async_copy(s, d, sem_g.at[j % DEPTH]).wait()
    for j in (CPT + 2, CPT + 3):
        drain_idx(tbase + j, j % DEPTH)
    for j in (CPT - 2, CPT - 1):
        s, d = sct_pair(j % DEPTH)
        pltpu.make_async_copy(s, d, sem_s.at[j % DEPTH]).wait()

    plsc.subcore_barrier()

    for k in range(nfull):
        off = row0 + k * EDGE_CHUNK
        pltpu.sync_copy(agg_sp.at[pl.ds(off, EDGE_CHUNK)],
                        agg_hbm.at[pl.ds(base_row + off, EDGE_CHUNK)])


def _spmm(x, idx_all, val_pad):
    mesh = plsc.VectorSubcoreMesh(core_axis_name="c", subcore_axis_name="s")
    return pl.kernel(
        _spmm_body,
        out_type=jax.ShapeDtypeStruct((N_PAD, HIDDEN), jnp.float32),
        mesh=mesh,
        scratch_types=[
            pltpu.VMEM_SHARED((HALF, HIDDEN), jnp.float32),
            pltpu.VMEM((DEPTH, EDGE_CHUNK, HIDDEN), jnp.float32),
            pltpu.VMEM((DEPTH, 2, EDGE_CHUNK), jnp.int32),
            pltpu.VMEM((DEPTH, EDGE_CHUNK), jnp.float32),
            pltpu.VMEM((DEPTH, EDGE_CHUNK), jnp.int32),
            pltpu.SemaphoreType.DMA((DEPTH,)),
            pltpu.SemaphoreType.DMA((DEPTH,)),
            pltpu.SemaphoreType.DMA((DEPTH,)),
        ],
        compiler_params=pltpu.CompilerParams(use_tc_tiling_on_sc=False),
    )(x, idx_all, val_pad)


def _out_proj_body(agg_ref, w_ref, out_ref):
    out_ref[...] = jnp.maximum(
        jnp.dot(agg_ref[...], w_ref[...], preferred_element_type=jnp.float32),
        0.0)


def _out_proj(agg, gcn_weight):
    return pl.pallas_call(
        _out_proj_body,
        grid=(GRID,),
        in_specs=[
            pl.BlockSpec((BLK, HIDDEN), lambda i: (i, 0)),
            pl.BlockSpec((HIDDEN, OUT_FEAT), lambda i: (0, 0)),
        ],
        out_specs=pl.BlockSpec((BLK, OUT_FEAT), lambda i: (i, 0)),
        out_shape=jax.ShapeDtypeStruct((N, OUT_FEAT), jnp.float32),
    )(agg, gcn_weight)


@jax.jit
def kernel(features, coordinates, adj_indices, adj_values, W_feat, b_feat,
           W_coord, b_coord, gamma, beta, gcn_weight):
    x = _embed_ln(features, coordinates, W_feat, b_feat, W_coord, b_coord,
                  gamma, beta)
    idx_all = jnp.zeros((2, E_PAD), jnp.int32)
    idx_all = idx_all.at[:, :E].set(adj_indices)
    val_pad = jnp.zeros((E_PAD,), jnp.float32).at[:E].set(adj_values)
    agg = _spmm(x, idx_all, val_pad)
    out = _out_proj(agg, gcn_weight)
    return out


# speedup vs baseline: 7.8371x; 1.6155x over previous
import jax
import jax.numpy as jnp
from jax import lax
from jax.experimental import pallas as pl
from jax.experimental.pallas import tpu as pltpu
from jax.experimental.pallas import tpu_sc as plsc

N = 50000
E = 800000
IN_FEAT = 128
HIDDEN = 64
OUT_FEAT = 128
HH = HIDDEN // 2

BLK = 1024
GRID = (N + BLK - 1) // BLK
N_PAD = GRID * BLK

NC = 2
NS = 16
ROWS_PER_TILE = N_PAD // NS
EDGE_CHUNK = 128
CPT = 396
DEPTH = 6
LOOK = 3
E_PAD = (NS * CPT + DEPTH) * EDGE_CHUNK


def _embed_ln_body(feat_ref, coord_ref, wf_ref, bf_ref, wc_ref, bc_ref,
                   gamma_ref, beta_ref, x_ref):
    f = feat_ref[...]
    fe = jnp.maximum(
        jnp.dot(f, wf_ref[...], preferred_element_type=jnp.float32)
        + bf_ref[...], 0.0)
    c = coord_ref[...]
    ce = jnp.maximum(
        c[:, 0:1] * wc_ref[0:1, :] + c[:, 1:2] * wc_ref[1:2, :] + bc_ref[...],
        0.0)
    x = fe + ce
    mu = jnp.mean(x, axis=-1, keepdims=True)
    var = jnp.mean((x - mu) ** 2, axis=-1, keepdims=True)
    y = (x - mu) * lax.rsqrt(var + 1e-5) * gamma_ref[...] + beta_ref[...]
    x_ref[...] = jnp.stack([y[:, :HH], y[:, HH:]])


def _embed_ln(features, coordinates, W_feat, b_feat, W_coord, b_coord,
              gamma, beta):
    return pl.pallas_call(
        _embed_ln_body,
        grid=(GRID,),
        in_specs=[
            pl.BlockSpec((BLK, IN_FEAT), lambda i: (i, 0)),
            pl.BlockSpec((BLK, 2), lambda i: (i, 0)),
            pl.BlockSpec((IN_FEAT, HIDDEN), lambda i: (0, 0)),
            pl.BlockSpec((1, HIDDEN), lambda i: (0, 0)),
            pl.BlockSpec((2, HIDDEN), lambda i: (0, 0)),
            pl.BlockSpec((1, HIDDEN), lambda i: (0, 0)),
            pl.BlockSpec((1, HIDDEN), lambda i: (0, 0)),
            pl.BlockSpec((1, HIDDEN), lambda i: (0, 0)),
        ],
        out_specs=pl.BlockSpec((2, BLK, HH), lambda i: (0, i, 0)),
        out_shape=jax.ShapeDtypeStruct((2, N, HH), jnp.float32),
    )(features, coordinates, W_feat, b_feat.reshape(1, HIDDEN),
      W_coord, b_coord.reshape(1, HIDDEN), gamma.reshape(1, HIDDEN),
      beta.reshape(1, HIDDEN))


def _spmm_body(x_hbm, idx_hbm, val_hbm, agg0_hbm, agg1_hbm,
               agg_sp, xb, ibuf, vbuf, rlb, sem_i, sem_g, sem_s):
    cid = lax.axis_index("c")
    sid = lax.axis_index("s")
    tbase = sid * CPT
    row0 = sid * ROWS_PER_TILE

    def _zrow(r, _):
        for d in range(HH // 16):
            xb[0, r, pl.ds(d * 16, 16)] = jnp.zeros((16,), jnp.float32)
        return 0
    lax.fori_loop(0, EDGE_CHUNK, _zrow, 0)
    nfull = ROWS_PER_TILE // EDGE_CHUNK
    rem = ROWS_PER_TILE - nfull * EDGE_CHUNK
    for k in range(nfull):
        pltpu.sync_copy(xb.at[0],
                        agg_sp.at[pl.ds(row0 + k * EDGE_CHUNK, EDGE_CHUNK)])
    pltpu.sync_copy(xb.at[0, pl.ds(0, rem)],
                    agg_sp.at[pl.ds(row0 + nfull * EDGE_CHUNK, rem)])
    plsc.subcore_barrier()

    def idx_pair(j, b):
        return idx_hbm.at[:, pl.ds(j * EDGE_CHUNK, EDGE_CHUNK)], ibuf.at[b]

    def val_pair(j, b):
        return val_hbm.at[pl.ds(j * EDGE_CHUNK, EDGE_CHUNK)], vbuf.at[b]

    def issue_idx(j, b):
        s, d = idx_pair(j, b)
        pltpu.async_copy(s, d, sem_i.at[b])
        s, d = val_pair(j, b)
        pltpu.async_copy(s, d, sem_i.at[b])

    def drain_idx(j, b):
        s, d = idx_pair(j, b)
        pltpu.make_async_copy(s, d, sem_i.at[b]).wait()
        s, d = val_pair(j, b)
        pltpu.make_async_copy(s, d, sem_i.at[b]).wait()

    def gat_pair(b):
        return x_hbm.at[cid].at[ibuf.at[b, 1]], xb.at[b]

    def sct_pair(b):
        return xb.at[b], agg_sp.at[rlb.at[b]]

    for b in range(DEPTH):
        issue_idx(tbase + b, b)
    for b in range(LOOK):
        drain_idx(tbase + b, b)
        s, d = gat_pair(b)
        pltpu.async_copy(s, d, sem_g.at[b])

    def compute_chunk(p):
        @plsc.parallel_loop(0, EDGE_CHUNK // 16, 1, unroll=2)
        def gbody(g):
            rlb[p, pl.ds(g * 16, 16)] = ibuf[p, 0, pl.ds(g * 16, 16)]
            w = vbuf[p, pl.ds(g * 16, 16)]
            for l in range(16):
                ws = w[l]
                e = g * 16 + l
                for dd in range(HH // 16):
                    xb[p, e, pl.ds(dd * 16, 16)] = (
                        xb[p, e, pl.ds(dd * 16, 16)] * ws)

    def pipe_body(i, _):
        for p in range(DEPTH):
            jl = i * DEPTH + p
            j = tbase + jl
            pg = (p + LOOK) % DEPTH
            drain_idx(j + LOOK, pg)
            @pl.when(jl >= LOOK)
            def _drain_scatter():
                s2, d2 = sct_pair(pg)
                pltpu.make_async_copy(s2, d2, sem_s.at[pg]).wait()
            s, d = gat_pair(pg)
            pltpu.async_copy(s, d, sem_g.at[pg])
            s, d = gat_pair(p)
            pltpu.make_async_copy(s, d, sem_g.at[p]).wait()
            compute_chunk(p)
            issue_idx(j + DEPTH, p)
            s, d = sct_pair(p)
            pltpu.async_copy(s, d, sem_s.at[p], add=True)
        return 0

    lax.fori_loop(0, CPT // DEPTH, pipe_body, 0)

    for j in range(CPT, CPT + LOOK):
        s, d = gat_pair(j % DEPTH)
        pltpu.make_async_copy(s, d, sem_g.at[j % DEPTH]).wait()
    for j in range(CPT + LOOK, CPT + DEPTH):
        drain_idx(tbase + j, j % DEPTH)
    for j in range(CPT - LOOK, CPT):
        s, d = sct_pair(j % DEPTH)
        pltpu.make_async_copy(s, d, sem_s.at[j % DEPTH]).wait()

    plsc.subcore_barrier()

    def flush(dst):
        for k in range(nfull):
            off = row0 + k * EDGE_CHUNK
            pltpu.sync_copy(agg_sp.at[pl.ds(off, EDGE_CHUNK)],
                            dst.at[pl.ds(off, EDGE_CHUNK)])
        off = row0 + nfull * EDGE_CHUNK
        pltpu.sync_copy(agg_sp.at[pl.ds(off, rem)], dst.at[pl.ds(off, rem)])

    @pl.when(cid == 0)
    def _flush0():
        flush(agg0_hbm)

    @pl.when(cid == 1)
    def _flush1():
        flush(agg1_hbm)


def _spmm(x, idx_all, val_pad):
    mesh = plsc.VectorSubcoreMesh(core_axis_name="c", subcore_axis_name="s")
    return pl.kernel(
        _spmm_body,
        out_type=(jax.ShapeDtypeStruct((N_PAD, HH), jnp.float32),
                  jax.ShapeDtypeStruct((N_PAD, HH), jnp.float32)),
        mesh=mesh,
        scratch_types=[
            pltpu.VMEM_SHARED((N_PAD, HH), jnp.float32),
            pltpu.VMEM((DEPTH, EDGE_CHUNK, HH), jnp.float32),
            pltpu.VMEM((DEPTH, 2, EDGE_CHUNK), jnp.int32),
            pltpu.VMEM((DEPTH, EDGE_CHUNK), jnp.float32),
            pltpu.VMEM((DEPTH, EDGE_CHUNK), jnp.int32),
            pltpu.SemaphoreType.DMA((DEPTH,)),
            pltpu.SemaphoreType.DMA((DEPTH,)),
            pltpu.SemaphoreType.DMA((DEPTH,)),
        ],
        compiler_params=pltpu.CompilerParams(use_tc_tiling_on_sc=False),
    )(x, idx_all, val_pad)


def _out_proj_body(agg0_ref, agg1_ref, w_ref, out_ref):
    acc = jnp.dot(agg0_ref[...], w_ref[0:HH, :],
                  preferred_element_type=jnp.float32)
    acc = acc + jnp.dot(agg1_ref[...], w_ref[HH:HIDDEN, :],
                        preferred_element_type=jnp.float32)
    out_ref[...] = jnp.maximum(acc, 0.0)


def _out_proj(agg0, agg1, gcn_weight):
    return pl.pallas_call(
        _out_proj_body,
        grid=(GRID,),
        in_specs=[
            pl.BlockSpec((BLK, HH), lambda i: (i, 0)),
            pl.BlockSpec((BLK, HH), lambda i: (i, 0)),
            pl.BlockSpec((HIDDEN, OUT_FEAT), lambda i: (0, 0)),
        ],
        out_specs=pl.BlockSpec((BLK, OUT_FEAT), lambda i: (i, 0)),
        out_shape=jax.ShapeDtypeStruct((N, OUT_FEAT), jnp.float32),
    )(agg0, agg1, gcn_weight)


@jax.jit
def kernel(features, coordinates, adj_indices, adj_values, W_feat, b_feat,
           W_coord, b_coord, gamma, beta, gcn_weight):
    x = _embed_ln(features, coordinates, W_feat, b_feat, W_coord, b_coord,
                  gamma, beta)
    idx_all = jnp.zeros((2, E_PAD), jnp.int32)
    idx_all = idx_all.at[:, :E].set(adj_indices)
    val_pad = jnp.zeros((E_PAD,), jnp.float32).at[:E].set(adj_values)
    agg0, agg1 = _spmm(x, idx_all, val_pad)
    out = _out_proj(agg0, agg1, gcn_weight)
    return out


# speedup vs baseline: 10.9718x; 1.4000x over previous
import jax
import jax.numpy as jnp
from jax import lax
from jax.experimental import pallas as pl
from jax.experimental.pallas import tpu as pltpu
from jax.experimental.pallas import tpu_sc as plsc

N = 50000
E = 800000
IN_FEAT = 128
HIDDEN = 64
OUT_FEAT = 128
HH = HIDDEN // 2

BLK = 1024
GRID = (N + BLK - 1) // BLK
N_PAD = GRID * BLK

NC = 2
NS = 16
ROWS_PER_TILE = N_PAD // NS
EDGE_CHUNK = 128
NCHUNKS = E // EDGE_CHUNK
CPT = 396
DEPTH = 6
LOOK = 3


def _embed_ln_body(feat_ref, coord_ref, wf_ref, bf_ref, wc_ref, bc_ref,
                   gamma_ref, beta_ref, x_ref):
    f = feat_ref[...]
    fe = jnp.maximum(
        jnp.dot(f, wf_ref[...], preferred_element_type=jnp.float32)
        + bf_ref[...], 0.0)
    c = coord_ref[...]
    ce = jnp.maximum(
        c[:, 0:1] * wc_ref[0:1, :] + c[:, 1:2] * wc_ref[1:2, :] + bc_ref[...],
        0.0)
    x = fe + ce
    mu = jnp.mean(x, axis=-1, keepdims=True)
    var = jnp.mean((x - mu) ** 2, axis=-1, keepdims=True)
    y = (x - mu) * lax.rsqrt(var + 1e-5) * gamma_ref[...] + beta_ref[...]
    x_ref[...] = jnp.stack([y[:, :HH], y[:, HH:]])


def _embed_ln(features, coordinates, W_feat, b_feat, W_coord, b_coord,
              gamma, beta):
    return pl.pallas_call(
        _embed_ln_body,
        grid=(GRID,),
        in_specs=[
            pl.BlockSpec((BLK, IN_FEAT), lambda i: (i, 0)),
            pl.BlockSpec((BLK, 2), lambda i: (i, 0)),
            pl.BlockSpec((IN_FEAT, HIDDEN), lambda i: (0, 0)),
            pl.BlockSpec((1, HIDDEN), lambda i: (0, 0)),
            pl.BlockSpec((2, HIDDEN), lambda i: (0, 0)),
            pl.BlockSpec((1, HIDDEN), lambda i: (0, 0)),
            pl.BlockSpec((1, HIDDEN), lambda i: (0, 0)),
            pl.BlockSpec((1, HIDDEN), lambda i: (0, 0)),
        ],
        out_specs=pl.BlockSpec((2, BLK, HH), lambda i: (0, i, 0)),
        out_shape=jax.ShapeDtypeStruct((2, N, HH), jnp.float32),
    )(features, coordinates, W_feat, b_feat.reshape(1, HIDDEN),
      W_coord, b_coord.reshape(1, HIDDEN), gamma.reshape(1, HIDDEN),
      beta.reshape(1, HIDDEN))


def _spmm_body(x_hbm, idx_hbm, val_hbm, agg0_hbm, agg1_hbm,
               agg_sp, xb, ibuf, vbuf, rlb, sem_i, sem_g, sem_s):
    cid = lax.axis_index("c")
    sid = lax.axis_index("s")
    tbase = sid * CPT
    row0 = sid * ROWS_PER_TILE

    def _zrow(r, _):
        for d in range(HH // 16):
            xb[0, r, pl.ds(d * 16, 16)] = jnp.zeros((16,), jnp.float32)
        return 0
    lax.fori_loop(0, EDGE_CHUNK, _zrow, 0)
    nfull = ROWS_PER_TILE // EDGE_CHUNK
    rem = ROWS_PER_TILE - nfull * EDGE_CHUNK
    for k in range(nfull):
        pltpu.sync_copy(xb.at[0],
                        agg_sp.at[pl.ds(row0 + k * EDGE_CHUNK, EDGE_CHUNK)])
    pltpu.sync_copy(xb.at[0, pl.ds(0, rem)],
                    agg_sp.at[pl.ds(row0 + nfull * EDGE_CHUNK, rem)])
    plsc.subcore_barrier()

    def idx_pair(j, b):
        jc = jnp.minimum(j, NCHUNKS - 1)
        return idx_hbm.at[:, pl.ds(jc * EDGE_CHUNK, EDGE_CHUNK)], ibuf.at[b]

    def val_pair(j, b):
        jc = jnp.minimum(j, NCHUNKS - 1)
        return val_hbm.at[pl.ds(jc * EDGE_CHUNK, EDGE_CHUNK)], vbuf.at[b]

    def issue_idx(j, b):
        s, d = idx_pair(j, b)
        pltpu.async_copy(s, d, sem_i.at[b])
        s, d = val_pair(j, b)
        pltpu.async_copy(s, d, sem_i.at[b])

    def drain_idx(j, b):
        s, d = idx_pair(j, b)
        pltpu.make_async_copy(s, d, sem_i.at[b]).wait()
        s, d = val_pair(j, b)
        pltpu.make_async_copy(s, d, sem_i.at[b]).wait()

    def gat_pair(b):
        return x_hbm.at[cid].at[ibuf.at[b, 1]], xb.at[b]

    def sct_pair(b):
        return xb.at[b], agg_sp.at[rlb.at[b]]

    for b in range(DEPTH):
        issue_idx(tbase + b, b)
    for b in range(LOOK):
        drain_idx(tbase + b, b)
        s, d = gat_pair(b)
        pltpu.async_copy(s, d, sem_g.at[b])

    def compute_chunk(p, wmask):
        @plsc.parallel_loop(0, EDGE_CHUNK // 16, 1, unroll=4)
        def gbody(g):
            rlb[p, pl.ds(g * 16, 16)] = ibuf[p, 0, pl.ds(g * 16, 16)]
            w = vbuf[p, pl.ds(g * 16, 16)] * wmask
            for l in range(16):
                ws = w[l]
                e = g * 16 + l
                for dd in range(HH // 16):
                    xb[p, e, pl.ds(dd * 16, 16)] = (
                        xb[p, e, pl.ds(dd * 16, 16)] * ws)

    def pipe_body(i, _):
        for p in range(DEPTH):
            jl = i * DEPTH + p
            j = tbase + jl
            pg = (p + LOOK) % DEPTH
            drain_idx(j + LOOK, pg)
            @pl.when(jl >= LOOK)
            def _drain_scatter():
                s2, d2 = sct_pair(pg)
                pltpu.make_async_copy(s2, d2, sem_s.at[pg]).wait()
            s, d = gat_pair(pg)
            pltpu.async_copy(s, d, sem_g.at[pg])
            s, d = gat_pair(p)
            pltpu.make_async_copy(s, d, sem_g.at[p]).wait()
            compute_chunk(p, jnp.where(j < NCHUNKS, 1.0, 0.0))
            issue_idx(j + DEPTH, p)
            s, d = sct_pair(p)
            pltpu.async_copy(s, d, sem_s.at[p], add=True)
        return 0

    lax.fori_loop(0, CPT // DEPTH, pipe_body, 0)

    for j in range(CPT, CPT + LOOK):
        s, d = gat_pair(j % DEPTH)
        pltpu.make_async_copy(s, d, sem_g.at[j % DEPTH]).wait()
    for j in range(CPT + LOOK, CPT + DEPTH):
        drain_idx(tbase + j, j % DEPTH)
    for j in range(CPT - LOOK, CPT):
        s, d = sct_pair(j % DEPTH)
        pltpu.make_async_copy(s, d, sem_s.at[j % DEPTH]).wait()

    plsc.subcore_barrier()

    def flush(dst):
        for k in range(nfull):
            off = row0 + k * EDGE_CHUNK
            pltpu.sync_copy(agg_sp.at[pl.ds(off, EDGE_CHUNK)],
                            dst.at[pl.ds(off, EDGE_CHUNK)])
        off = row0 + nfull * EDGE_CHUNK
        pltpu.sync_copy(agg_sp.at[pl.ds(off, rem)], dst.at[pl.ds(off, rem)])

    @pl.when(cid == 0)
    def _flush0():
        flush(agg0_hbm)

    @pl.when(cid == 1)
    def _flush1():
        flush(agg1_hbm)


def _spmm(x, idx_all, val_pad):
    mesh = plsc.VectorSubcoreMesh(core_axis_name="c", subcore_axis_name="s")
    return pl.kernel(
        _spmm_body,
        out_type=(jax.ShapeDtypeStruct((N_PAD, HH), jnp.float32),
                  jax.ShapeDtypeStruct((N_PAD, HH), jnp.float32)),
        mesh=mesh,
        scratch_types=[
            pltpu.VMEM_SHARED((N_PAD, HH), jnp.float32),
            pltpu.VMEM((DEPTH, EDGE_CHUNK, HH), jnp.float32),
            pltpu.VMEM((DEPTH, 2, EDGE_CHUNK), jnp.int32),
            pltpu.VMEM((DEPTH, EDGE_CHUNK), jnp.float32),
            pltpu.VMEM((DEPTH, EDGE_CHUNK), jnp.int32),
            pltpu.SemaphoreType.DMA((DEPTH,)),
            pltpu.SemaphoreType.DMA((DEPTH,)),
            pltpu.SemaphoreType.DMA((DEPTH,)),
        ],
        compiler_params=pltpu.CompilerParams(use_tc_tiling_on_sc=False),
    )(x, idx_all, val_pad)


def _out_proj_body(agg0_ref, agg1_ref, w_ref, out_ref):
    acc = jnp.dot(agg0_ref[...], w_ref[0:HH, :],
                  preferred_element_type=jnp.float32)
    acc = acc + jnp.dot(agg1_ref[...], w_ref[HH:HIDDEN, :],
                        preferred_element_type=jnp.float32)
    out_ref[...] = jnp.maximum(acc, 0.0)


def _out_proj(agg0, agg1, gcn_weight):
    return pl.pallas_call(
        _out_proj_body,
        grid=(GRID,),
        in_specs=[
            pl.BlockSpec((BLK, HH), lambda i: (i, 0)),
            pl.BlockSpec((BLK, HH), lambda i: (i, 0)),
            pl.BlockSpec((HIDDEN, OUT_FEAT), lambda i: (0, 0)),
        ],
        out_specs=pl.BlockSpec((BLK, OUT_FEAT), lambda i: (i, 0)),
        out_shape=jax.ShapeDtypeStruct((N, OUT_FEAT), jnp.float32),
    )(agg0, agg1, gcn_weight)


@jax.jit
def kernel(features, coordinates, adj_indices, adj_values, W_feat, b_feat,
           W_coord, b_coord, gamma, beta, gcn_weight):
    x = _embed_ln(features, coordinates, W_feat, b_feat, W_coord, b_coord,
                  gamma, beta)
    agg0, agg1 = _spmm(x, adj_indices, adj_values)
    out = _out_proj(agg0, agg1, gcn_weight)
    return out
